# baseline (speedup 1.0000x reference)
import functools

import jax
import jax.numpy as jnp
from jax.experimental import pallas as pl

IN_CHANNELS = 256
HIDDEN_DIM = 128
NUM_EXPERTS = 4


def _gate_kernel(x_ref, w1_ref, b1_ref, w2_ref, b2_ref, out_ref):
    b = pl.program_id(0)
    x = x_ref[0]
    pooled = jnp.sum(x, axis=1) * (1.0 / x.shape[1])
    pooled = pooled.reshape(1, -1)
    h = jnp.dot(pooled, w1_ref[...], preferred_element_type=jnp.float32)
    h = h + b1_ref[...]
    h = 0.5 * h * (1.0 + jax.lax.erf(h * 0.7071067811865476))
    logits = jnp.dot(h, w2_ref[...], preferred_element_type=jnp.float32)
    logits = logits + b2_ref[...]
    m = jnp.max(logits, axis=-1, keepdims=True)
    e = jnp.exp(logits - m)
    weights = e / jnp.sum(e, axis=-1, keepdims=True)
    out_ref[pl.ds(b, 1), :] = weights


@jax.jit
def kernel(img_emb, W1, b1, W2, b2):
    B, C, H, W = img_emb.shape
    x = img_emb.reshape(B, C, H * W)
    b1r = b1.reshape(1, HIDDEN_DIM)
    b2r = b2.reshape(1, NUM_EXPERTS)
    out = pl.pallas_call(
        _gate_kernel,
        grid=(B,),
        in_specs=[
            pl.BlockSpec((1, C, H * W), lambda b: (b, 0, 0)),
            pl.BlockSpec((C, HIDDEN_DIM), lambda b: (0, 0)),
            pl.BlockSpec((1, HIDDEN_DIM), lambda b: (0, 0)),
            pl.BlockSpec((HIDDEN_DIM, NUM_EXPERTS), lambda b: (0, 0)),
            pl.BlockSpec((1, NUM_EXPERTS), lambda b: (0, 0)),
        ],
        out_specs=pl.BlockSpec((B, NUM_EXPERTS), lambda b: (0, 0)),
        out_shape=jax.ShapeDtypeStruct((B, NUM_EXPERTS), jnp.float32),
    )(x, W1, b1r, W2, b2r)
    return out
